# baseline (speedup 1.0000x reference)
import functools

import jax
import jax.numpy as jnp
from jax import lax
from jax.experimental import pallas as pl
from jax.experimental.pallas import tpu as pltpu
from jax.experimental.pallas import tpu_sc as plsc

N_NODES = 10000
N_EDGES = 160000
IN_DIM = 256
OUT_DIM = 256

NC = 2
NS = 16
L = 16
HALF = OUT_DIM // NC

EDGES_PER_TEC = N_EDGES // NS
CHUNK = 80
N_CHUNKS = EDGES_PER_TEC // CHUNK
STAGE = 25
N_STAGE = N_CHUNKS // STAGE
N_PAIRS = (STAGE - 1) // 2
OUT_CHUNK = 16
N_OUT_CHUNKS = N_NODES // OUT_CHUNK
OUT_PER_TEC = -(-N_OUT_CHUNKS // NS)

ROW_BLOCK = 2000


def _matmul_body(x_ref, w_ref, o_ref):
    o_ref[0] = jnp.dot(x_ref[...], w_ref[...], preferred_element_type=jnp.float32)


def _matmul_split(x, W):
    grid = (NC, N_NODES // ROW_BLOCK)
    return pl.pallas_call(
        _matmul_body,
        grid=grid,
        in_specs=[
            pl.BlockSpec((ROW_BLOCK, IN_DIM), lambda c, r: (r, 0)),
            pl.BlockSpec((IN_DIM, HALF), lambda c, r: (0, c)),
        ],
        out_specs=pl.BlockSpec((1, ROW_BLOCK, HALF), lambda c, r: (c, r, 0)),
        out_shape=jax.ShapeDtypeStruct((NC, N_NODES, HALF), jnp.float32),
    )(x, W)


def _sc_aggregate(h2, src, dst, vals):
    mesh = plsc.VectorSubcoreMesh(core_axis_name="c", subcore_axis_name="s")
    src4 = src.reshape(NS, N_STAGE, STAGE, CHUNK)
    dst4 = dst.reshape(NS, N_STAGE, STAGE, CHUNK)
    vals4 = vals.reshape(NS, N_STAGE, STAGE, CHUNK)

    @functools.partial(
        pl.kernel,
        out_type=jax.ShapeDtypeStruct((N_NODES, OUT_DIM), jnp.float32),
        mesh=mesh,
        scratch_types=[
            pltpu.VMEM((STAGE, CHUNK), jnp.int32),
            pltpu.VMEM((STAGE, CHUNK), jnp.int32),
            pltpu.VMEM((STAGE, CHUNK), jnp.float32),
            pltpu.VMEM((CHUNK, HALF), jnp.float32),
            pltpu.VMEM((CHUNK, HALF), jnp.float32),
            pltpu.VMEM((OUT_CHUNK, HALF), jnp.float32),
            pltpu.VMEM_SHARED((N_NODES, HALF), jnp.float32),
            pltpu.SemaphoreType.DMA,
            pltpu.SemaphoreType.DMA,
        ],
    )
    def k(h2_hbm, src_hbm, dst_hbm, vals_hbm, out_hbm,
          src_v, dst_v, vals_v, rows_a, rows_b, buf_v, acc, gsem, ssem):
        c = lax.axis_index("c")
        s = lax.axis_index("s")

        with jax.named_scope("acc_zero"):
            zeros = jnp.zeros((L,), jnp.float32)

            @pl.loop(0, OUT_CHUNK)
            def _(i):
                for r in range(HALF // L):
                    buf_v[i, pl.ds(r * L, L)] = zeros

            @pl.loop(0, OUT_PER_TEC)
            def _(j):
                t = j * NS + s

                @pl.when(t < N_OUT_CHUNKS)
                def _():
                    pltpu.sync_copy(buf_v, acc.at[pl.ds(t * OUT_CHUNK, OUT_CHUNK)])

            plsc.subcore_barrier()

        def g_fire(rows, j):
            pltpu.async_copy(h2_hbm.at[c].at[src_v.at[j]], rows, gsem)

        def g_wait(rows, j):
            pltpu.make_async_copy(h2_hbm.at[c].at[src_v.at[j]], rows, gsem).wait()

        def s_fire(rows, j):
            pltpu.async_copy(rows, acc.at[dst_v.at[j]], ssem, add=True)

        def s_wait(rows, j):
            pltpu.make_async_copy(rows, acc.at[dst_v.at[j]], ssem).wait()

        def scale(rows, j):
            for g in range(CHUNK // L):
                vv = vals_v[j, pl.ds(g * L, L)]
                for e in range(L):
                    v = vv[e]
                    for r in range(HALF // L):
                        sl = pl.ds(r * L, L)
                        rows[g * L + e, sl] = rows[g * L + e, sl] * v

        with jax.named_scope("edge_pipeline"):
            @pl.loop(0, N_STAGE)
            def _(b):
                pltpu.sync_copy(src_hbm.at[s, b], src_v)
                pltpu.sync_copy(dst_hbm.at[s, b], dst_v)
                pltpu.sync_copy(vals_hbm.at[s, b], vals_v)

                g_fire(rows_a, 0)

                @pl.loop(0, N_PAIRS)
                def _(p):
                    ja = 2 * p
                    jb = 2 * p + 1

                    @pl.when(p > 0)
                    def _():
                        s_wait(rows_b, jb - 2)

                    g_fire(rows_b, jb)
                    g_wait(rows_a, ja)
                    scale(rows_a, ja)
                    s_fire(rows_a, ja)
                    g_wait(rows_b, jb)
                    scale(rows_b, jb)
                    s_fire(rows_b, jb)
                    s_wait(rows_a, ja)
                    g_fire(rows_a, ja + 2)

                s_wait(rows_b, 2 * N_PAIRS - 1)
                g_wait(rows_a, STAGE - 1)
                scale(rows_a, STAGE - 1)
                s_fire(rows_a, STAGE - 1)
                s_wait(rows_a, STAGE - 1)

            plsc.subcore_barrier()

        with jax.named_scope("relu_copy_out"):
            @pl.loop(0, OUT_PER_TEC)
            def _(j):
                t = j * NS + s

                @pl.when(t < N_OUT_CHUNKS)
                def _():
                    r0 = t * OUT_CHUNK
                    pltpu.sync_copy(acc.at[pl.ds(r0, OUT_CHUNK)], buf_v)

                    @pl.loop(0, OUT_CHUNK)
                    def _(i):
                        for r in range(HALF // L):
                            sl = pl.ds(r * L, L)
                            buf_v[i, sl] = jnp.maximum(buf_v[i, sl], 0.0)

                    pltpu.sync_copy(
                        buf_v, out_hbm.at[pl.ds(r0, OUT_CHUNK), pl.ds(c * HALF, HALF)])

    return k(h2, src4, dst4, vals4)


def kernel(x, edge_index, edge_vals, W):
    src = edge_index[0].astype(jnp.int32)
    dst = edge_index[1].astype(jnp.int32)
    h2 = _matmul_split(x, W)
    return _sc_aggregate(h2, src, dst, edge_vals)


# speedup vs baseline: 1.2188x; 1.2188x over previous
import functools

import jax
import jax.numpy as jnp
from jax import lax
from jax.experimental import pallas as pl
from jax.experimental.pallas import tpu as pltpu
from jax.experimental.pallas import tpu_sc as plsc

N_NODES = 10000
N_EDGES = 160000
IN_DIM = 256
OUT_DIM = 256

NC = 2
NS = 16
L = 16
HALF = OUT_DIM // NC

EDGES_PER_TEC = N_EDGES // NS
CHUNK = 80
N_CHUNKS = EDGES_PER_TEC // CHUNK
STAGE = 25
N_STAGE = N_CHUNKS // STAGE
N_PAIRS = (STAGE - 1) // 2
OUT_CHUNK = 40
N_OUT_CHUNKS = N_NODES // OUT_CHUNK
OUT_PER_TEC = -(-N_OUT_CHUNKS // NS)

ROW_BLOCK = 2000


def _matmul_body(x_ref, w_ref, o_ref):
    o_ref[0] = jnp.dot(x_ref[...], w_ref[...], preferred_element_type=jnp.float32)


def _matmul_split(x, W):
    grid = (NC, N_NODES // ROW_BLOCK)
    return pl.pallas_call(
        _matmul_body,
        grid=grid,
        in_specs=[
            pl.BlockSpec((ROW_BLOCK, IN_DIM), lambda c, r: (r, 0)),
            pl.BlockSpec((IN_DIM, HALF), lambda c, r: (0, c)),
        ],
        out_specs=pl.BlockSpec((1, ROW_BLOCK, HALF), lambda c, r: (c, r, 0)),
        out_shape=jax.ShapeDtypeStruct((NC, N_NODES, HALF), jnp.float32),
    )(x, W)


def _sc_aggregate(h2, src, dst, vals):
    mesh = plsc.VectorSubcoreMesh(core_axis_name="c", subcore_axis_name="s")
    src4 = src.reshape(NS, N_STAGE, STAGE, CHUNK)
    dst4 = dst.reshape(NS, N_STAGE, STAGE, CHUNK)
    vals4 = vals.reshape(NS, N_STAGE, STAGE, CHUNK)

    @functools.partial(
        pl.kernel,
        out_type=jax.ShapeDtypeStruct((N_NODES, OUT_DIM), jnp.float32),
        mesh=mesh,
        scratch_types=[
            pltpu.VMEM((STAGE, CHUNK), jnp.int32),
            pltpu.VMEM((STAGE, CHUNK), jnp.int32),
            pltpu.VMEM((STAGE, CHUNK), jnp.float32),
            pltpu.VMEM((CHUNK, HALF), jnp.float32),
            pltpu.VMEM((CHUNK, HALF), jnp.float32),
            pltpu.VMEM((OUT_CHUNK, HALF), jnp.float32),
            pltpu.VMEM_SHARED((N_NODES, HALF), jnp.float32),
            pltpu.SemaphoreType.DMA,
            pltpu.SemaphoreType.DMA,
        ],
    )
    def k(h2_hbm, src_hbm, dst_hbm, vals_hbm, out_hbm,
          src_v, dst_v, vals_v, rows_a, rows_b, buf_v, acc, gsem, ssem):
        c = lax.axis_index("c")
        s = lax.axis_index("s")

        zeros = jnp.zeros((L,), jnp.float32)

        @pl.loop(0, OUT_CHUNK)
        def _(i):
            for r in range(HALF // L):
                buf_v[i, pl.ds(r * L, L)] = zeros

        @pl.loop(0, OUT_PER_TEC)
        def _(j):
            t = j * NS + s

            @pl.when(t < N_OUT_CHUNKS)
            def _():
                pltpu.sync_copy(buf_v, acc.at[pl.ds(t * OUT_CHUNK, OUT_CHUNK)])

        plsc.subcore_barrier()

        def g_fire(rows, j):
            pltpu.async_copy(h2_hbm.at[c].at[src_v.at[j]], rows, gsem)

        def g_wait(rows, j):
            pltpu.make_async_copy(h2_hbm.at[c].at[src_v.at[j]], rows, gsem).wait()

        def s_fire(rows, j):
            pltpu.async_copy(rows, acc.at[dst_v.at[j]], ssem, add=True)

        def s_wait(rows, j):
            pltpu.make_async_copy(rows, acc.at[dst_v.at[j]], ssem).wait()

        def scale(rows, j):
            @pl.loop(0, CHUNK // L)
            def _(g):
                vv = vals_v[j, pl.ds(g * L, L)]
                for e in range(L):
                    v = vv[e]
                    for r in range(HALF // L):
                        sl = pl.ds(r * L, L)
                        rows[g * L + e, sl] = rows[g * L + e, sl] * v

        @pl.loop(0, N_STAGE)
        def _(b):
            pltpu.sync_copy(src_hbm.at[s, b], src_v)
            pltpu.sync_copy(dst_hbm.at[s, b], dst_v)
            pltpu.sync_copy(vals_hbm.at[s, b], vals_v)

            g_fire(rows_a, 0)

            @pl.loop(0, N_PAIRS)
            def _(p):
                ja = 2 * p
                jb = 2 * p + 1
                g_fire(rows_b, jb)
                g_wait(rows_a, ja)
                scale(rows_a, ja)
                s_fire(rows_a, ja)
                g_wait(rows_b, jb)
                scale(rows_b, jb)
                s_fire(rows_b, jb)
                s_wait(rows_a, ja)
                g_fire(rows_a, ja + 2)
                s_wait(rows_b, jb)

            g_wait(rows_a, STAGE - 1)
            scale(rows_a, STAGE - 1)
            s_fire(rows_a, STAGE - 1)
            s_wait(rows_a, STAGE - 1)

        plsc.subcore_barrier()

        @pl.loop(0, OUT_PER_TEC)
        def _(j):
            t = j * NS + s

            @pl.when(t < N_OUT_CHUNKS)
            def _():
                r0 = t * OUT_CHUNK
                pltpu.sync_copy(acc.at[pl.ds(r0, OUT_CHUNK)], buf_v)

                @pl.loop(0, OUT_CHUNK)
                def _(i):
                    for r in range(HALF // L):
                        sl = pl.ds(r * L, L)
                        buf_v[i, sl] = jnp.maximum(buf_v[i, sl], 0.0)

                pltpu.sync_copy(
                    buf_v, out_hbm.at[pl.ds(r0, OUT_CHUNK), pl.ds(c * HALF, HALF)])

    return k(h2, src4, dst4, vals4)


def kernel(x, edge_index, edge_vals, W):
    src = edge_index[0].astype(jnp.int32)
    dst = edge_index[1].astype(jnp.int32)
    h2 = _matmul_split(x, W)
    return _sc_aggregate(h2, src, dst, edge_vals)


# speedup vs baseline: 1.2623x; 1.0357x over previous
import functools

import jax
import jax.numpy as jnp
from jax import lax
from jax.experimental import pallas as pl
from jax.experimental.pallas import tpu as pltpu
from jax.experimental.pallas import tpu_sc as plsc

N_NODES = 10000
N_EDGES = 160000
IN_DIM = 256
OUT_DIM = 256

NC = 2
NS = 16
L = 16
HALF = OUT_DIM // NC

EDGES_PER_TEC = N_EDGES // NS
CHUNK = 80
N_CHUNKS = EDGES_PER_TEC // CHUNK
STAGE = 25
N_STAGE = N_CHUNKS // STAGE
N_PAIRS = (STAGE - 1) // 2
OUT_CHUNK = CHUNK
N_OUT_CHUNKS = N_NODES // OUT_CHUNK
OUT_PER_TEC = -(-N_OUT_CHUNKS // NS)

ROW_BLOCK = 2000


def _matmul_body(x_ref, w_ref, o_ref):
    o_ref[0] = jnp.dot(x_ref[...], w_ref[...], preferred_element_type=jnp.float32)


def _matmul_split(x, W):
    grid = (NC, N_NODES // ROW_BLOCK)
    return pl.pallas_call(
        _matmul_body,
        grid=grid,
        in_specs=[
            pl.BlockSpec((ROW_BLOCK, IN_DIM), lambda c, r: (r, 0)),
            pl.BlockSpec((IN_DIM, HALF), lambda c, r: (0, c)),
        ],
        out_specs=pl.BlockSpec((1, ROW_BLOCK, HALF), lambda c, r: (c, r, 0)),
        out_shape=jax.ShapeDtypeStruct((NC, N_NODES, HALF), jnp.float32),
    )(x, W)


def _sc_aggregate(h2, src, dst, vals):
    mesh = plsc.VectorSubcoreMesh(core_axis_name="c", subcore_axis_name="s")
    src4 = src.reshape(NS, N_STAGE, STAGE, CHUNK)
    dst4 = dst.reshape(NS, N_STAGE, STAGE, CHUNK)
    vals4 = vals.reshape(NS, N_STAGE, STAGE, CHUNK)

    @functools.partial(
        pl.kernel,
        out_type=jax.ShapeDtypeStruct((N_NODES, OUT_DIM), jnp.float32),
        mesh=mesh,
        scratch_types=[
            pltpu.VMEM((STAGE, CHUNK), jnp.int32),
            pltpu.VMEM((STAGE, CHUNK), jnp.int32),
            pltpu.VMEM((STAGE, CHUNK), jnp.float32),
            pltpu.VMEM((CHUNK, HALF), jnp.float32),
            pltpu.VMEM((CHUNK, HALF), jnp.float32),
            pltpu.VMEM_SHARED((N_NODES, HALF), jnp.float32),
            pltpu.SemaphoreType.DMA,
            pltpu.SemaphoreType.DMA,
        ],
    )
    def k(h2_hbm, src_hbm, dst_hbm, vals_hbm, out_hbm,
          src_v, dst_v, vals_v, rows_a, rows_b, acc, gsem, ssem):
        c = lax.axis_index("c")
        s = lax.axis_index("s")

        zeros = jnp.zeros((L,), jnp.float32)

        @pl.loop(0, OUT_CHUNK)
        def _(i):
            for r in range(HALF // L):
                rows_a[i, pl.ds(r * L, L)] = zeros

        @pl.loop(0, OUT_PER_TEC)
        def _(j):
            t = j * NS + s

            @pl.when(t < N_OUT_CHUNKS)
            def _():
                pltpu.sync_copy(rows_a, acc.at[pl.ds(t * OUT_CHUNK, OUT_CHUNK)])

        plsc.subcore_barrier()

        def g_fire(rows, j):
            pltpu.async_copy(h2_hbm.at[c].at[src_v.at[j]], rows, gsem)

        def g_wait(rows, j):
            pltpu.make_async_copy(h2_hbm.at[c].at[src_v.at[j]], rows, gsem).wait()

        def s_fire(rows, j):
            pltpu.async_copy(rows, acc.at[dst_v.at[j]], ssem, add=True)

        def s_wait(rows, j):
            pltpu.make_async_copy(rows, acc.at[dst_v.at[j]], ssem).wait()

        def scale(rows, j):
            @pl.loop(0, CHUNK // L)
            def _(g):
                vv = vals_v[j, pl.ds(g * L, L)]
                for e in range(L):
                    v = vv[e]
                    for r in range(HALF // L):
                        sl = pl.ds(r * L, L)
                        rows[g * L + e, sl] = rows[g * L + e, sl] * v

        @pl.loop(0, N_STAGE)
        def _(b):
            c1 = pltpu.async_copy(src_hbm.at[s, b], src_v, gsem)
            c2 = pltpu.async_copy(dst_hbm.at[s, b], dst_v, gsem)
            c3 = pltpu.async_copy(vals_hbm.at[s, b], vals_v, gsem)
            c1.wait()
            c2.wait()
            c3.wait()

            g_fire(rows_a, 0)

            @pl.loop(0, N_PAIRS)
            def _(p):
                ja = 2 * p
                jb = 2 * p + 1
                g_fire(rows_b, jb)
                g_wait(rows_a, ja)
                scale(rows_a, ja)
                s_fire(rows_a, ja)
                g_wait(rows_b, jb)
                scale(rows_b, jb)
                s_fire(rows_b, jb)
                s_wait(rows_a, ja)
                g_fire(rows_a, ja + 2)
                s_wait(rows_b, jb)

            g_wait(rows_a, STAGE - 1)
            scale(rows_a, STAGE - 1)
            s_fire(rows_a, STAGE - 1)
            s_wait(rows_a, STAGE - 1)

        plsc.subcore_barrier()

        @pl.loop(0, OUT_PER_TEC)
        def _(j):
            t = j * NS + s

            @pl.when(t < N_OUT_CHUNKS)
            def _():
                r0 = t * OUT_CHUNK
                pltpu.sync_copy(acc.at[pl.ds(r0, OUT_CHUNK)], rows_a)

                @pl.loop(0, OUT_CHUNK)
                def _(i):
                    for r in range(HALF // L):
                        sl = pl.ds(r * L, L)
                        rows_a[i, sl] = jnp.maximum(rows_a[i, sl], 0.0)

                pltpu.sync_copy(
                    rows_a, out_hbm.at[pl.ds(r0, OUT_CHUNK), pl.ds(c * HALF, HALF)])

    return k(h2, src4, dst4, vals4)


def kernel(x, edge_index, edge_vals, W):
    src = edge_index[0].astype(jnp.int32)
    dst = edge_index[1].astype(jnp.int32)
    h2 = _matmul_split(x, W)
    return _sc_aggregate(h2, src, dst, edge_vals)


# speedup vs baseline: 1.2678x; 1.0043x over previous
import functools

import jax
import jax.numpy as jnp
from jax import lax
from jax.experimental import pallas as pl
from jax.experimental.pallas import tpu as pltpu
from jax.experimental.pallas import tpu_sc as plsc

N_NODES = 10000
N_EDGES = 160000
IN_DIM = 256
OUT_DIM = 256

NC = 2
NS = 16
L = 16
HALF = OUT_DIM // NC

EDGES_PER_TEC = N_EDGES // NS
CHUNK = 80
N_CHUNKS = EDGES_PER_TEC // CHUNK
STAGE = 25
N_STAGE = N_CHUNKS // STAGE
N_PAIRS = (STAGE - 1) // 2
OUT_CHUNK = CHUNK
N_OUT_CHUNKS = N_NODES // OUT_CHUNK
OUT_PER_TEC = -(-N_OUT_CHUNKS // NS)

ROW_BLOCK = 2000


def _matmul_body(x_ref, w_ref, o_ref):
    o_ref[0] = jnp.dot(x_ref[...], w_ref[...], preferred_element_type=jnp.float32)


def _matmul_split(x, W):
    grid = (NC, N_NODES // ROW_BLOCK)
    return pl.pallas_call(
        _matmul_body,
        grid=grid,
        in_specs=[
            pl.BlockSpec((ROW_BLOCK, IN_DIM), lambda c, r: (r, 0)),
            pl.BlockSpec((IN_DIM, HALF), lambda c, r: (0, c)),
        ],
        out_specs=pl.BlockSpec((1, ROW_BLOCK, HALF), lambda c, r: (c, r, 0)),
        out_shape=jax.ShapeDtypeStruct((NC, N_NODES, HALF), jnp.float32),
    )(x, W)


def _sc_aggregate(h2, src, dst, vals):
    mesh = plsc.VectorSubcoreMesh(core_axis_name="c", subcore_axis_name="s")
    src4 = src.reshape(NS, N_STAGE, STAGE, CHUNK)
    dst4 = dst.reshape(NS, N_STAGE, STAGE, CHUNK)
    vals4 = vals.reshape(NS, N_STAGE, STAGE, CHUNK)

    @functools.partial(
        pl.kernel,
        out_type=jax.ShapeDtypeStruct((N_NODES, OUT_DIM), jnp.float32),
        mesh=mesh,
        scratch_types=[
            pltpu.VMEM((STAGE, CHUNK), jnp.int32),
            pltpu.VMEM((STAGE, CHUNK), jnp.int32),
            pltpu.VMEM((STAGE, CHUNK), jnp.float32),
            pltpu.VMEM((CHUNK, HALF), jnp.float32),
            pltpu.VMEM((CHUNK, HALF), jnp.float32),
            pltpu.VMEM_SHARED((N_NODES, HALF), jnp.float32),
            pltpu.SemaphoreType.DMA,
            pltpu.SemaphoreType.DMA,
        ],
    )
    def k(h2_hbm, src_hbm, dst_hbm, vals_hbm, out_hbm,
          src_v, dst_v, vals_v, rows_a, rows_b, acc, gsem, ssem):
        c = lax.axis_index("c")
        s = lax.axis_index("s")

        zeros = jnp.zeros((L,), jnp.float32)

        @pl.loop(0, OUT_CHUNK)
        def _(i):
            for r in range(HALF // L):
                rows_a[i, pl.ds(r * L, L)] = zeros

        @pl.loop(0, OUT_PER_TEC)
        def _(j):
            t = j * NS + s

            @pl.when(t < N_OUT_CHUNKS)
            def _():
                pltpu.async_copy(
                    rows_a, acc.at[pl.ds(t * OUT_CHUNK, OUT_CHUNK)], gsem)

        @pl.loop(0, OUT_PER_TEC)
        def _(j):
            t = j * NS + s

            @pl.when(t < N_OUT_CHUNKS)
            def _():
                pltpu.make_async_copy(
                    rows_a, acc.at[pl.ds(t * OUT_CHUNK, OUT_CHUNK)], gsem).wait()

        plsc.subcore_barrier()

        def g_fire(rows, j):
            pltpu.async_copy(h2_hbm.at[c].at[src_v.at[j]], rows, gsem)

        def g_wait(rows, j):
            pltpu.make_async_copy(h2_hbm.at[c].at[src_v.at[j]], rows, gsem).wait()

        def s_fire(rows, j):
            pltpu.async_copy(rows, acc.at[dst_v.at[j]], ssem, add=True)

        def s_wait(rows, j):
            pltpu.make_async_copy(rows, acc.at[dst_v.at[j]], ssem).wait()

        def scale(rows, j):
            @pl.loop(0, CHUNK // L)
            def _(g):
                vv = vals_v[j, pl.ds(g * L, L)]
                for e in range(L):
                    v = vv[e]
                    for r in range(HALF // L):
                        sl = pl.ds(r * L, L)
                        rows[g * L + e, sl] = rows[g * L + e, sl] * v

        @pl.loop(0, N_STAGE)
        def _(b):
            c1 = pltpu.async_copy(src_hbm.at[s, b], src_v, gsem)
            c2 = pltpu.async_copy(dst_hbm.at[s, b], dst_v, gsem)
            c3 = pltpu.async_copy(vals_hbm.at[s, b], vals_v, gsem)
            c1.wait()
            c2.wait()
            c3.wait()

            g_fire(rows_a, 0)

            @pl.loop(0, N_PAIRS)
            def _(p):
                ja = 2 * p
                jb = 2 * p + 1
                g_fire(rows_b, jb)
                g_wait(rows_a, ja)
                scale(rows_a, ja)
                s_fire(rows_a, ja)
                g_wait(rows_b, jb)
                scale(rows_b, jb)
                s_fire(rows_b, jb)
                s_wait(rows_a, ja)
                g_fire(rows_a, ja + 2)
                s_wait(rows_b, jb)

            g_wait(rows_a, STAGE - 1)
            scale(rows_a, STAGE - 1)
            s_fire(rows_a, STAGE - 1)
            s_wait(rows_a, STAGE - 1)

        plsc.subcore_barrier()

        @pl.loop(0, OUT_PER_TEC)
        def _(j):
            t = j * NS + s

            @pl.when(t < N_OUT_CHUNKS)
            def _():
                r0 = t * OUT_CHUNK
                pltpu.sync_copy(acc.at[pl.ds(r0, OUT_CHUNK)], rows_a)

                @pl.loop(0, OUT_CHUNK)
                def _(i):
                    for r in range(HALF // L):
                        sl = pl.ds(r * L, L)
                        rows_a[i, sl] = jnp.maximum(rows_a[i, sl], 0.0)

                pltpu.sync_copy(
                    rows_a, out_hbm.at[pl.ds(r0, OUT_CHUNK), pl.ds(c * HALF, HALF)])

    return k(h2, src4, dst4, vals4)


def kernel(x, edge_index, edge_vals, W):
    src = edge_index[0].astype(jnp.int32)
    dst = edge_index[1].astype(jnp.int32)
    h2 = _matmul_split(x, W)
    return _sc_aggregate(h2, src, dst, edge_vals)
